# Initial kernel scaffold; baseline (speedup 1.0000x reference)
import jax
import jax.numpy as jnp
from jax.experimental import pallas as pl


def kernel(x, table, W, b, mark_absent):
    raise NotImplementedError("write your pallas kernel here")


# speedup vs baseline: 40.5947x; 40.5947x over previous
import functools

import jax
import jax.numpy as jnp
from jax import lax
from jax.experimental import pallas as pl
from jax.experimental.pallas import tpu as pltpu
from jax.experimental.pallas import tpu_sc as plsc

_BS = 4096
_NOBJ = 21
_P = 26
_E = 32
_ROWS = _BS * _NOBJ
_NIDX = _ROWS * _P

_NC = 2
_NS = 16
_NW = _NC * _NS
_PER_W = _NIDX // _NW
_CHUNK = 728
_NCHUNK = _PER_W // _CHUNK

assert _PER_W * _NW == _NIDX
assert _CHUNK * _NCHUNK == _PER_W and _CHUNK % 8 == 0


@functools.partial(
    pl.kernel,
    out_type=jax.ShapeDtypeStruct((_NIDX, _E), jnp.float32),
    mesh=plsc.VectorSubcoreMesh(core_axis_name="c", subcore_axis_name="s"),
    scratch_types=[
        pltpu.VMEM((_CHUNK,), jnp.int32),
        pltpu.VMEM((_CHUNK, _E), jnp.float32),
        pltpu.SemaphoreType.DMA,
    ],
    compiler_params=pltpu.CompilerParams(use_tc_tiling_on_sc=False),
)
def _sc_gather(idx_hbm, table_hbm, out_hbm, idx_v, rows_v, sem):
    wid = lax.axis_index("s") * _NC + lax.axis_index("c")
    base = wid * _PER_W

    def body(i, carry):
        off = base + i * _CHUNK
        pltpu.sync_copy(idx_hbm.at[pl.ds(off, _CHUNK)], idx_v)
        pltpu.async_copy(table_hbm.at[idx_v], rows_v, sem).wait()
        pltpu.sync_copy(rows_v, out_hbm.at[pl.ds(off, _CHUNK)])
        return carry

    lax.fori_loop(0, _NCHUNK, body, 0)


_RB = 1024


def _tc_proj(g_ref, xs_ref, wt_ref, b_ref, ma_ref, y_ref, m_ref):
    y = jnp.dot(g_ref[...], wt_ref[...], preferred_element_type=jnp.float32)
    y = y + b_ref[...]
    pad = jnp.sum(xs_ref[...], axis=1, keepdims=True) == 0
    y_ref[...] = jnp.where(pad, ma_ref[...], y)
    m_ref[...] = pad.astype(jnp.int32)


def kernel(x, table, W, b, mark_absent):
    idx_flat = x.reshape(_NIDX)
    gathered = _sc_gather(idx_flat, table)

    g2 = gathered.reshape(_ROWS, _P * _E)
    xs = x.reshape(_ROWS, _P)
    wt = W.T
    y, m = pl.pallas_call(
        _tc_proj,
        grid=(_ROWS // _RB,),
        in_specs=[
            pl.BlockSpec((_RB, _P * _E), lambda i: (i, 0)),
            pl.BlockSpec((_RB, _P), lambda i: (i, 0)),
            pl.BlockSpec((_P * _E, _E), lambda i: (0, 0)),
            pl.BlockSpec((1, _E), lambda i: (0, 0)),
            pl.BlockSpec((1, _E), lambda i: (0, 0)),
        ],
        out_specs=[
            pl.BlockSpec((_RB, _E), lambda i: (i, 0)),
            pl.BlockSpec((_RB, 1), lambda i: (i, 0)),
        ],
        out_shape=[
            jax.ShapeDtypeStruct((_ROWS, _E), jnp.float32),
            jax.ShapeDtypeStruct((_ROWS, 1), jnp.int32),
        ],
    )(g2, xs, wt, b.reshape(1, _E), mark_absent.reshape(1, _E))

    obj_emb = y.reshape(_BS, _NOBJ, _E)
    padding = m.reshape(_BS, _NOBJ) != 0
    return obj_emb, padding


# speedup vs baseline: 43.9435x; 1.0825x over previous
import functools

import jax
import jax.numpy as jnp
from jax import lax
from jax.experimental import pallas as pl
from jax.experimental.pallas import tpu as pltpu
from jax.experimental.pallas import tpu_sc as plsc

_BS = 4096
_NOBJ = 21
_P = 26
_E = 32
_ROWS = _BS * _NOBJ
_NIDX = _ROWS * _P

_NC = 2
_NS = 16
_NW = _NC * _NS
_PER_W = _NIDX // _NW
_CHUNK = 1456
_NCHUNK = _PER_W // _CHUNK

assert _PER_W * _NW == _NIDX
assert _CHUNK * _NCHUNK == _PER_W and _CHUNK % 8 == 0
assert _NCHUNK % 2 == 0 and _NCHUNK >= 4


@functools.partial(
    pl.kernel,
    out_type=jax.ShapeDtypeStruct((_NIDX, _E), jnp.float32),
    mesh=plsc.VectorSubcoreMesh(core_axis_name="c", subcore_axis_name="s"),
    scratch_types=[
        pltpu.VMEM((2, _CHUNK), jnp.int32),
        pltpu.VMEM((2, _CHUNK, _E), jnp.float32),
        pltpu.SemaphoreType.DMA,
        pltpu.SemaphoreType.DMA,
        pltpu.SemaphoreType.DMA,
        pltpu.SemaphoreType.DMA,
    ],
    compiler_params=pltpu.CompilerParams(use_tc_tiling_on_sc=False),
)
def _sc_gather(idx_hbm, table_hbm, out_hbm, idx_v, rows_v, g0, g1, s0, s1):
    wid = lax.axis_index("s") * _NC + lax.axis_index("c")
    base = wid * _PER_W
    gsem = (g0, g1)
    ssem = (s0, s1)

    def gather_start(i, b):
        off = base + i * _CHUNK
        pltpu.sync_copy(idx_hbm.at[pl.ds(off, _CHUNK)], idx_v.at[b])
        pltpu.async_copy(table_hbm.at[idx_v.at[b]], rows_v.at[b], gsem[b])

    def gather_wait(b):
        pltpu.make_async_copy(table_hbm.at[idx_v.at[b]], rows_v.at[b], gsem[b]).wait()

    def store_start(i, b):
        off = base + i * _CHUNK
        pltpu.async_copy(rows_v.at[b], out_hbm.at[pl.ds(off, _CHUNK)], ssem[b])

    def store_wait(i, b):
        off = base + i * _CHUNK
        pltpu.make_async_copy(rows_v.at[b], out_hbm.at[pl.ds(off, _CHUNK)], ssem[b]).wait()

    gather_start(0, 0)
    gather_start(1, 1)

    def pair(j, carry):
        for b in range(2):
            i = 2 * j + b
            gather_wait(b)
            store_start(i, b)
            store_wait(i, b)
            gather_start(i + 2, b)
        return carry

    lax.fori_loop(0, (_NCHUNK - 2) // 2, pair, 0)

    for b in range(2):
        i = _NCHUNK - 2 + b
        gather_wait(b)
        store_start(i, b)
    for b in range(2):
        store_wait(_NCHUNK - 2 + b, b)


_RB = 1024


def _tc_proj(g_ref, xs_ref, wt_ref, b_ref, ma_ref, y_ref, m_ref):
    y = jnp.dot(g_ref[...], wt_ref[...], preferred_element_type=jnp.float32)
    y = y + b_ref[...]
    pad = jnp.sum(xs_ref[...], axis=1, keepdims=True) == 0
    y_ref[...] = jnp.where(pad, ma_ref[...], y)
    m_ref[...] = pad.astype(jnp.int32)


def kernel(x, table, W, b, mark_absent):
    idx_flat = x.reshape(_NIDX)
    gathered = _sc_gather(idx_flat, table)

    g2 = gathered.reshape(_ROWS, _P * _E)
    xs = x.reshape(_ROWS, _P)
    wt = W.T
    y, m = pl.pallas_call(
        _tc_proj,
        grid=(_ROWS // _RB,),
        in_specs=[
            pl.BlockSpec((_RB, _P * _E), lambda i: (i, 0)),
            pl.BlockSpec((_RB, _P), lambda i: (i, 0)),
            pl.BlockSpec((_P * _E, _E), lambda i: (0, 0)),
            pl.BlockSpec((1, _E), lambda i: (0, 0)),
            pl.BlockSpec((1, _E), lambda i: (0, 0)),
        ],
        out_specs=[
            pl.BlockSpec((_RB, _E), lambda i: (i, 0)),
            pl.BlockSpec((_RB, 1), lambda i: (i, 0)),
        ],
        out_shape=[
            jax.ShapeDtypeStruct((_ROWS, _E), jnp.float32),
            jax.ShapeDtypeStruct((_ROWS, 1), jnp.int32),
        ],
    )(g2, xs, wt, b.reshape(1, _E), mark_absent.reshape(1, _E))

    obj_emb = y.reshape(_BS, _NOBJ, _E)
    padding = m.reshape(_BS, _NOBJ) != 0
    return obj_emb, padding
